# initial kernel scaffold (unmeasured)
import jax
import jax.numpy as jnp
from jax import lax
from jax.experimental import pallas as pl
from jax.experimental.pallas import tpu as pltpu

N_DEV = 4
SQ = 256
SKV = 4096
DMODEL = 1024
DH = 128
N_HEADS = 8
D = N_HEADS * DH
SCALE = 0.08838834764831843


def _body(
    x_ref, wq_ref, wo_ref, k_hbm, v_hbm, out_ref,
    stage, k_bf, v_bf, attn, comm,
    load_sems, send_sems, recv_sems,
):
    my = lax.axis_index("i")
    left = lax.rem(my + (N_DEV - 1), N_DEV)
    right = lax.rem(my + 1, N_DEV)

    barrier = pltpu.get_barrier_semaphore()
    for nbr in (left, right):
        pl.semaphore_signal(
            barrier, inc=1,
            device_id=(nbr,), device_id_type=pl.DeviceIdType.MESH,
        )
    pl.semaphore_wait(barrier, 2)

    col0 = my * D
    k_load = pltpu.make_async_copy(
        k_hbm.at[:, pl.ds(col0, D)], stage, load_sems.at[0]
    )
    k_load.start()

    x_bf = x_ref[...].astype(jnp.bfloat16)
    wq_bf = wq_ref[...].astype(jnp.bfloat16)
    q = lax.dot_general(
        x_bf, wq_bf, (((1,), (0,)), ((), ())),
        preferred_element_type=jnp.float32,
    )
    q_bf = q.astype(jnp.bfloat16)

    k_load.wait()
    k_bf[...] = stage[...].astype(jnp.bfloat16)
    v_load = pltpu.make_async_copy(
        v_hbm.at[:, pl.ds(col0, D)], stage, load_sems.at[1]
    )
    v_load.start()
    v_load.wait()
    v_bf[...] = stage[...].astype(jnp.bfloat16)

    for h in range(N_HEADS):
        qh = q_bf[:, h * DH:(h + 1) * DH]
        kh = k_bf[:, h * DH:(h + 1) * DH]
        vh = v_bf[:, h * DH:(h + 1) * DH]
        s = lax.dot_general(
            qh, kh, (((1,), (1,)), ((), ())),
            preferred_element_type=jnp.float32,
        ) * SCALE
        m = jnp.max(s, axis=1, keepdims=True)
        p = jnp.exp(s - m)
        l = jnp.sum(p, axis=1, keepdims=True)
        o = lax.dot_general(
            p.astype(jnp.bfloat16), vh, (((1,), (0,)), ((), ())),
            preferred_element_type=jnp.float32,
        )
        attn[:, h * DH:(h + 1) * DH] = (o / l).astype(jnp.bfloat16)

    wo_bf = wo_ref[...].astype(jnp.bfloat16)
    partial = lax.dot_general(
        attn[...], wo_bf, (((1,), (0,)), ((), ())),
        preferred_element_type=jnp.float32,
    )
    comm[0] = partial
    out_ref[...] = partial

    for hop in range(N_DEV - 1):
        rdma = pltpu.make_async_remote_copy(
            src_ref=comm.at[hop],
            dst_ref=comm.at[hop + 1],
            send_sem=send_sems.at[hop],
            recv_sem=recv_sems.at[hop + 1],
            device_id=(right,),
            device_id_type=pl.DeviceIdType.MESH,
        )
        rdma.start()
        rdma.wait()
        out_ref[...] += comm[hop + 1]


def kernel(x, Wq, Wo, K_ext, V_ext):
    x2 = x.reshape(SQ, DMODEL)
    k2 = K_ext.reshape(SKV, 4 * D)
    v2 = V_ext.reshape(SKV, 4 * D)
    out = pl.pallas_call(
        _body,
        out_shape=jax.ShapeDtypeStruct((SQ, DMODEL), jnp.float32),
        in_specs=[
            pl.BlockSpec(memory_space=pltpu.VMEM),
            pl.BlockSpec(memory_space=pltpu.VMEM),
            pl.BlockSpec(memory_space=pltpu.VMEM),
            pl.BlockSpec(memory_space=pltpu.ANY),
            pl.BlockSpec(memory_space=pltpu.ANY),
        ],
        out_specs=pl.BlockSpec(memory_space=pltpu.VMEM),
        scratch_shapes=[
            pltpu.VMEM((SKV, D), jnp.float32),
            pltpu.VMEM((SKV, D), jnp.bfloat16),
            pltpu.VMEM((SKV, D), jnp.bfloat16),
            pltpu.VMEM((SQ, D), jnp.bfloat16),
            pltpu.VMEM((N_DEV, SQ, DMODEL), jnp.float32),
            pltpu.SemaphoreType.DMA((2,)),
            pltpu.SemaphoreType.DMA((N_DEV,)),
            pltpu.SemaphoreType.DMA((N_DEV,)),
        ],
        compiler_params=pltpu.CompilerParams(collective_id=0),
    )(x2, Wq, Wo, k2, v2)
    return out.reshape(1, SQ, DMODEL)


# baseline (device time: 174691 ns/iter reference)
import jax
import jax.numpy as jnp
from jax import lax
from jax.experimental import pallas as pl
from jax.experimental.pallas import tpu as pltpu

N_DEV = 4
SQ = 256
SKV = 4096
DMODEL = 1024
DH = 128
N_HEADS = 8
D = N_HEADS * DH
SCALE = 0.08838834764831843


def _body(
    x_ref, wq_ref, wo_ref, k_hbm, v_hbm, out_ref,
    stage, k_bf, v_bf, attn, comm,
    load_sems, send_sems, recv_sems,
):
    my = lax.axis_index("i")
    left = lax.rem(my + (N_DEV - 1), N_DEV)
    right = lax.rem(my + 1, N_DEV)

    barrier = pltpu.get_barrier_semaphore()
    for nbr in (left, right):
        pl.semaphore_signal(
            barrier, inc=1,
            device_id=(nbr,), device_id_type=pl.DeviceIdType.MESH,
        )
    pl.semaphore_wait(barrier, 2)

    col0 = my * D
    k_load = pltpu.make_async_copy(
        k_hbm.at[:, pl.ds(col0, D)], stage, load_sems.at[0]
    )
    k_load.start()

    x_bf = x_ref[...].astype(jnp.bfloat16)
    wq_bf = wq_ref[...].astype(jnp.bfloat16)
    q = lax.dot_general(
        x_bf, wq_bf, (((1,), (0,)), ((), ())),
        preferred_element_type=jnp.float32,
    )
    q_bf = q.astype(jnp.bfloat16)

    k_load.wait()
    k_bf[...] = stage[...].astype(jnp.bfloat16)
    v_load = pltpu.make_async_copy(
        v_hbm.at[:, pl.ds(col0, D)], stage, load_sems.at[1]
    )
    v_load.start()
    v_load.wait()
    v_bf[...] = stage[...].astype(jnp.bfloat16)

    for h in range(N_HEADS):
        qh = q_bf[:, h * DH:(h + 1) * DH]
        kh = k_bf[:, h * DH:(h + 1) * DH]
        vh = v_bf[:, h * DH:(h + 1) * DH]
        s = lax.dot_general(
            qh, kh, (((1,), (1,)), ((), ())),
            preferred_element_type=jnp.float32,
        ) * SCALE
        m = jnp.max(s, axis=1, keepdims=True)
        p = jnp.exp(s - m)
        l = jnp.sum(p, axis=1, keepdims=True)
        o = lax.dot_general(
            p.astype(jnp.bfloat16), vh, (((1,), (0,)), ((), ())),
            preferred_element_type=jnp.float32,
        )
        attn[:, h * DH:(h + 1) * DH] = (o / l).astype(jnp.bfloat16)

    wo_bf = wo_ref[...].astype(jnp.bfloat16)
    partial = lax.dot_general(
        attn[...], wo_bf, (((1,), (0,)), ((), ())),
        preferred_element_type=jnp.float32,
    )
    comm[0] = partial
    out_ref[...] = partial

    for hop in range(N_DEV - 1):
        rdma = pltpu.make_async_remote_copy(
            src_ref=comm.at[hop],
            dst_ref=comm.at[hop + 1],
            send_sem=send_sems.at[hop],
            recv_sem=recv_sems.at[hop + 1],
            device_id=(right,),
            device_id_type=pl.DeviceIdType.MESH,
        )
        rdma.start()
        rdma.wait()
        out_ref[...] += comm[hop + 1]


def kernel(x, Wq, Wo, K_ext, V_ext):
    x2 = x.reshape(SQ, DMODEL)
    k2 = K_ext.reshape(SKV, 4 * D)
    v2 = V_ext.reshape(SKV, 4 * D)
    out = pl.pallas_call(
        _body,
        out_shape=jax.ShapeDtypeStruct((SQ, DMODEL), jnp.float32),
        in_specs=[
            pl.BlockSpec(memory_space=pltpu.VMEM),
            pl.BlockSpec(memory_space=pltpu.VMEM),
            pl.BlockSpec(memory_space=pltpu.VMEM),
            pl.BlockSpec(memory_space=pl.ANY),
            pl.BlockSpec(memory_space=pl.ANY),
        ],
        out_specs=pl.BlockSpec(memory_space=pltpu.VMEM),
        scratch_shapes=[
            pltpu.VMEM((SKV, D), jnp.float32),
            pltpu.VMEM((SKV, D), jnp.bfloat16),
            pltpu.VMEM((SKV, D), jnp.bfloat16),
            pltpu.VMEM((SQ, D), jnp.bfloat16),
            pltpu.VMEM((N_DEV, SQ, DMODEL), jnp.float32),
            pltpu.SemaphoreType.DMA((2,)),
            pltpu.SemaphoreType.DMA((N_DEV,)),
            pltpu.SemaphoreType.DMA((N_DEV,)),
        ],
        compiler_params=pltpu.CompilerParams(
            collective_id=0,
            vmem_limit_bytes=100 * 1024 * 1024,
        ),
    )(x2, Wq, Wo, k2, v2)
    return out.reshape(1, SQ, DMODEL)


# device time: 62594 ns/iter; 2.7909x vs baseline; 2.7909x over previous
import jax
import jax.numpy as jnp
from jax import lax
from jax.experimental import pallas as pl
from jax.experimental.pallas import tpu as pltpu

N_DEV = 4
SQ = 256
SKV = 4096
DMODEL = 1024
DH = 128
N_HEADS = 8
D = N_HEADS * DH
CH = 1024
N_CHUNKS = SKV // CH
SCALE = 0.08838834764831843


def _body(
    x_ref, wq_ref, wo_ref, k_hbm, v_hbm, out_ref,
    stage, k_bf, v_bf, attn, comm,
    load_sems, send_sems, recv_sems,
):
    my = lax.axis_index("i")
    left = lax.rem(my + (N_DEV - 1), N_DEV)
    right = lax.rem(my + 1, N_DEV)

    barrier = pltpu.get_barrier_semaphore()
    for nbr in (left, right):
        pl.semaphore_signal(
            barrier, inc=1,
            device_id=(nbr,), device_id_type=pl.DeviceIdType.MESH,
        )
    pl.semaphore_wait(barrier, 2)

    h0 = my * N_HEADS
    steps = [(k_hbm, k_bf, c) for c in range(N_CHUNKS)] + [
        (v_hbm, v_bf, c) for c in range(N_CHUNKS)
    ]

    def _load(s):
        hbm, _, c = steps[s]
        return pltpu.make_async_copy(
            hbm.at[0, pl.ds(c * CH, CH), pl.ds(h0, N_HEADS), :],
            stage.at[s % 2],
            load_sems.at[s % 2],
        )

    _load(0).start()
    _load(1).start()

    x_bf = x_ref[...].astype(jnp.bfloat16)
    wq_bf = wq_ref[...].astype(jnp.bfloat16)
    q = lax.dot_general(
        x_bf, wq_bf, (((1,), (0,)), ((), ())),
        preferred_element_type=jnp.float32,
    )
    q_bf = q.astype(jnp.bfloat16)

    for s in range(len(steps)):
        _load(s).wait()
        _, dst, c = steps[s]
        dst[c * CH:(c + 1) * CH, :] = (
            stage[s % 2].astype(jnp.bfloat16).reshape(CH, D)
        )
        if s + 2 < len(steps):
            _load(s + 2).start()

    for h in range(N_HEADS):
        qh = q_bf[:, h * DH:(h + 1) * DH]
        kh = k_bf[:, h * DH:(h + 1) * DH]
        vh = v_bf[:, h * DH:(h + 1) * DH]
        s = lax.dot_general(
            qh, kh, (((1,), (1,)), ((), ())),
            preferred_element_type=jnp.float32,
        ) * SCALE
        m = jnp.max(s, axis=1, keepdims=True)
        p = jnp.exp(s - m)
        l = jnp.sum(p, axis=1, keepdims=True)
        o = lax.dot_general(
            p.astype(jnp.bfloat16), vh, (((1,), (0,)), ((), ())),
            preferred_element_type=jnp.float32,
        )
        attn[:, h * DH:(h + 1) * DH] = (o / l).astype(jnp.bfloat16)

    wo_bf = wo_ref[...].astype(jnp.bfloat16)
    partial = lax.dot_general(
        attn[...], wo_bf, (((1,), (0,)), ((), ())),
        preferred_element_type=jnp.float32,
    )
    comm[0] = partial.astype(jnp.bfloat16)
    out_ref[...] = partial

    for hop in range(N_DEV - 1):
        rdma = pltpu.make_async_remote_copy(
            src_ref=comm.at[hop],
            dst_ref=comm.at[hop + 1],
            send_sem=send_sems.at[hop],
            recv_sem=recv_sems.at[hop + 1],
            device_id=(right,),
            device_id_type=pl.DeviceIdType.MESH,
        )
        rdma.start()
        rdma.wait()
        out_ref[...] += comm[hop + 1].astype(jnp.float32)


def kernel(x, Wq, Wo, K_ext, V_ext):
    x2 = x.reshape(SQ, DMODEL)
    out = pl.pallas_call(
        _body,
        out_shape=jax.ShapeDtypeStruct((SQ, DMODEL), jnp.float32),
        in_specs=[
            pl.BlockSpec(memory_space=pltpu.VMEM),
            pl.BlockSpec(memory_space=pltpu.VMEM),
            pl.BlockSpec(memory_space=pltpu.VMEM),
            pl.BlockSpec(memory_space=pl.ANY),
            pl.BlockSpec(memory_space=pl.ANY),
        ],
        out_specs=pl.BlockSpec(memory_space=pltpu.VMEM),
        scratch_shapes=[
            pltpu.VMEM((2, CH, N_HEADS, DH), jnp.float32),
            pltpu.VMEM((SKV, D), jnp.bfloat16),
            pltpu.VMEM((SKV, D), jnp.bfloat16),
            pltpu.VMEM((SQ, D), jnp.bfloat16),
            pltpu.VMEM((N_DEV, SQ, DMODEL), jnp.bfloat16),
            pltpu.SemaphoreType.DMA((2,)),
            pltpu.SemaphoreType.DMA((N_DEV,)),
            pltpu.SemaphoreType.DMA((N_DEV,)),
        ],
        compiler_params=pltpu.CompilerParams(
            collective_id=0,
            vmem_limit_bytes=100 * 1024 * 1024,
        ),
    )(x2, Wq, Wo, K_ext, V_ext)
    return out.reshape(1, SQ, DMODEL)


# device time: 46205 ns/iter; 3.7808x vs baseline; 1.3547x over previous
import jax
import jax.numpy as jnp
from jax import lax
from jax.experimental import pallas as pl
from jax.experimental.pallas import tpu as pltpu

N_DEV = 4
SQ = 256
SKV = 4096
DMODEL = 1024
DH = 128
N_HEADS = 8
D = N_HEADS * DH
CH = 1024
N_CHUNKS = SKV // CH
SCALE = 0.08838834764831843
_RING = True


def _body(
    x_ref, wq_ref, wo_ref, k_hbm, v_hbm, out_ref,
    stage, kv_bf, attn, comm,
    load_sems, send_sems, recv_sems,
):
    my = lax.axis_index("i")
    left = lax.rem(my + (N_DEV - 1), N_DEV)
    right = lax.rem(my + 1, N_DEV)

    barrier = pltpu.get_barrier_semaphore()
    for nbr in (left, right):
        pl.semaphore_signal(
            barrier, inc=1,
            device_id=(nbr,), device_id_type=pl.DeviceIdType.MESH,
        )
    pl.semaphore_wait(barrier, 2)

    h0 = my * N_HEADS

    def _load(s):
        hbm = k_hbm if s % 2 == 0 else v_hbm
        c = s // 2
        return pltpu.make_async_copy(
            hbm.at[0, pl.ds(c * CH, CH), pl.ds(h0, N_HEADS), :],
            stage.at[s % 4],
            load_sems.at[s % 4],
        )

    for s in range(min(4, 2 * N_CHUNKS)):
        _load(s).start()

    x_bf = x_ref[...].astype(jnp.bfloat16)
    wq_bf = wq_ref[...].astype(jnp.bfloat16)
    q_bf = (
        lax.dot_general(
            x_bf, wq_bf, (((1,), (0,)), ((), ())),
            preferred_element_type=jnp.float32,
        ) * SCALE
    ).astype(jnp.bfloat16)

    l = [None] * N_HEADS
    acc = [None] * N_HEADS

    for c in range(N_CHUNKS):
        _load(2 * c).wait()
        kv_bf[0] = stage[(2 * c) % 4].astype(jnp.bfloat16).reshape(CH, D)
        _load(2 * c + 1).wait()
        kv_bf[1] = stage[(2 * c + 1) % 4].astype(jnp.bfloat16).reshape(CH, D)
        if 2 * c + 5 < 2 * N_CHUNKS:
            _load(2 * c + 4).start()
            _load(2 * c + 5).start()

        for h in range(N_HEADS):
            hs = slice(h * DH, (h + 1) * DH)
            s_ch = lax.dot_general(
                q_bf[:, hs], kv_bf[0, :, hs], (((1,), (1,)), ((), ())),
                preferred_element_type=jnp.float32,
            )
            p = jnp.exp(s_ch)
            l_c = jnp.sum(p, axis=1, keepdims=True)
            o_c = lax.dot_general(
                p.astype(jnp.bfloat16), kv_bf[1, :, hs],
                (((1,), (0,)), ((), ())),
                preferred_element_type=jnp.float32,
            )
            if c == 0:
                l[h], acc[h] = l_c, o_c
            else:
                l[h] = l[h] + l_c
                acc[h] = acc[h] + o_c

    for h in range(N_HEADS):
        hs = slice(h * DH, (h + 1) * DH)
        attn[:, hs] = (acc[h] / l[h]).astype(jnp.bfloat16)

    wo_bf = wo_ref[...].astype(jnp.bfloat16)
    partial = lax.dot_general(
        attn[...], wo_bf, (((1,), (0,)), ((), ())),
        preferred_element_type=jnp.float32,
    )
    comm[0] = partial.astype(jnp.bfloat16)
    out_ref[...] = partial

    if not _RING:
        return
    p1 = jnp.bitwise_xor(my, 1)
    p2 = jnp.bitwise_xor(my, 2)
    ex_a = pltpu.make_async_remote_copy(
        src_ref=comm.at[0],
        dst_ref=comm.at[1],
        send_sem=send_sems.at[0],
        recv_sem=recv_sems.at[1],
        device_id=(p1,),
        device_id_type=pl.DeviceIdType.MESH,
    )
    ex_a.start()
    ex_a.wait()
    sum_a = partial + comm[1].astype(jnp.float32)
    comm[2] = sum_a.astype(jnp.bfloat16)
    ex_b = pltpu.make_async_remote_copy(
        src_ref=comm.at[2],
        dst_ref=comm.at[3],
        send_sem=send_sems.at[2],
        recv_sem=recv_sems.at[3],
        device_id=(p2,),
        device_id_type=pl.DeviceIdType.MESH,
    )
    ex_b.start()
    ex_b.wait()
    out_ref[...] = sum_a + comm[3].astype(jnp.float32)


def kernel(x, Wq, Wo, K_ext, V_ext):
    x2 = x.reshape(SQ, DMODEL)
    out = pl.pallas_call(
        _body,
        out_shape=jax.ShapeDtypeStruct((SQ, DMODEL), jnp.float32),
        in_specs=[
            pl.BlockSpec(memory_space=pltpu.MemorySpace.VMEM),
            pl.BlockSpec(memory_space=pltpu.MemorySpace.VMEM),
            pl.BlockSpec(memory_space=pltpu.MemorySpace.VMEM),
            pl.BlockSpec(memory_space=pl.ANY),
            pl.BlockSpec(memory_space=pl.ANY),
        ],
        out_specs=pl.BlockSpec(memory_space=pltpu.MemorySpace.VMEM),
        scratch_shapes=[
            pltpu.VMEM((4, CH, N_HEADS, DH), jnp.float32),
            pltpu.VMEM((2, CH, D), jnp.bfloat16),
            pltpu.VMEM((SQ, D), jnp.bfloat16),
            pltpu.VMEM((N_DEV, SQ, DMODEL), jnp.bfloat16),
            pltpu.SemaphoreType.DMA((4,)),
            pltpu.SemaphoreType.DMA((N_DEV,)),
            pltpu.SemaphoreType.DMA((N_DEV,)),
        ],
        compiler_params=pltpu.CompilerParams(
            collective_id=0,
            vmem_limit_bytes=100 * 1024 * 1024,
        ),
    )(x2, Wq, Wo, K_ext, V_ext)
    return out.reshape(1, SQ, DMODEL)
